# baseline (speedup 1.0000x reference)
import functools

import jax
import jax.numpy as jnp
from jax import lax
from jax.experimental import pallas as pl
from jax.experimental.pallas import tpu as pltpu
from jax.experimental.pallas import tpu_sc as plsc

_N = 10000
_D = 128
_DW = _D // 2
_E = 320000
_NW = 32
_EPW = _E // _NW
_G = 80
_NG = _EPW // _G


def _mlp_block(x_ref, w_ref, b_ref, o_ref):
    y = jnp.dot(x_ref[...], w_ref[...], preferred_element_type=jnp.float32)
    o_ref[...] = jnp.maximum(y + b_ref[...], 0.0).astype(jnp.bfloat16)


def _transform(x, w, b, bl=2000):
    n, d = x.shape
    return pl.pallas_call(
        _mlp_block,
        grid=(n // bl,),
        in_specs=[
            pl.BlockSpec((bl, d), lambda i: (i, 0)),
            pl.BlockSpec((d, d), lambda i: (0, 0)),
            pl.BlockSpec((1, d), lambda i: (0, 0)),
        ],
        out_specs=pl.BlockSpec((bl, d), lambda i: (i, 0)),
        out_shape=jax.ShapeDtypeStruct((n, d), jnp.bfloat16),
    )(x, w, b.reshape(1, d))


def _edge_values(u_tab, v_tab, src, dst, ev):
    mesh = plsc.VectorSubcoreMesh(core_axis_name="c", subcore_axis_name="s")

    @functools.partial(
        pl.kernel,
        mesh=mesh,
        out_type=jax.ShapeDtypeStruct((_E,), jnp.float32),
        compiler_params=pltpu.CompilerParams(
            needs_layout_passes=False, use_tc_tiling_on_sc=False),
        scratch_types=[
            pltpu.VMEM((_EPW,), jnp.int32),
            pltpu.VMEM((_EPW,), jnp.int32),
            pltpu.VMEM((_EPW,), jnp.float32),
            pltpu.VMEM((_EPW,), jnp.float32),
            pltpu.VMEM((3, _G, _DW), jnp.float32),
            pltpu.VMEM((3, _G, _DW), jnp.float32),
            pltpu.SemaphoreType.DMA,
            pltpu.SemaphoreType.DMA,
            pltpu.SemaphoreType.DMA,
            pltpu.SemaphoreType.DMA,
            pltpu.SemaphoreType.DMA,
            pltpu.SemaphoreType.DMA,
        ],
    )
    def body(u_hbm, v_hbm, src_hbm, dst_hbm, ev_hbm, out_hbm,
             src_v, dst_v, ev_v, out_v, u_rows, v_rows,
             su0, sv0, su1, sv1, su2, sv2):
        wid = lax.axis_index("s") * 2 + lax.axis_index("c")
        base = wid * _EPW
        pltpu.sync_copy(src_hbm.at[pl.ds(base, _EPW)], src_v)
        pltpu.sync_copy(dst_hbm.at[pl.ds(base, _EPW)], dst_v)
        pltpu.sync_copy(ev_hbm.at[pl.ds(base, _EPW)], ev_v)

        lanes = lax.iota(jnp.int32, 16)
        sems = ((su0, sv0), (su1, sv1), (su2, sv2))

        _H = _G // 2

        def issue(g, b):
            gb = g * _G
            pltpu.async_copy(u_hbm.at[src_v.at[pl.ds(gb, _H)]],
                             u_rows.at[b].at[pl.ds(0, _H)], sems[b][0])
            pltpu.async_copy(u_hbm.at[src_v.at[pl.ds(gb + _H, _H)]],
                             u_rows.at[b].at[pl.ds(_H, _H)], sems[b][0])
            pltpu.async_copy(v_hbm.at[dst_v.at[pl.ds(gb, _H)]],
                             v_rows.at[b].at[pl.ds(0, _H)], sems[b][1])
            pltpu.async_copy(v_hbm.at[dst_v.at[pl.ds(gb + _H, _H)]],
                             v_rows.at[b].at[pl.ds(_H, _H)], sems[b][1])

        def compute(g, b):
            gb = g * _G
            for h in range(2):
                pltpu.make_async_copy(
                    u_hbm.at[src_v.at[pl.ds(gb + h * _H, _H)]],
                    u_rows.at[b].at[pl.ds(h * _H, _H)], sems[b][0]).wait()
                pltpu.make_async_copy(
                    v_hbm.at[dst_v.at[pl.ds(gb + h * _H, _H)]],
                    v_rows.at[b].at[pl.ds(h * _H, _H)], sems[b][1]).wait()

            def subgroup(sg, c):
                sgb = sg * 16
                d2 = jnp.zeros((16,), jnp.float32)
                for k in range(16):
                    e = sgb + k
                    acc = jnp.zeros((16,), jnp.float32)
                    for j in range(_DW // 16):
                        uw = u_rows[b, e, pl.ds(j * 16, 16)]
                        vw = v_rows[b, e, pl.ds(j * 16, 16)]
                        ub = plsc.bitcast(uw, jnp.bfloat16)
                        vb = plsc.bitcast(vw, jnp.bfloat16)
                        du = ub - vb
                        lo, hi = plsc.unpack(
                            du, format=plsc.PackFormat.INTERLEAVED)
                        acc = acc + lo * lo + hi * hi
                    d2 = jnp.where(lanes == k, jnp.sum(acc), d2)
                d2c = jnp.maximum(d2, 1e-30)
                bi = lax.bitcast_convert_type(d2c, jnp.int32)
                bi = 0x5F3759DF - lax.shift_right_arithmetic(bi, 1)
                y = lax.bitcast_convert_type(bi, jnp.float32)
                for _ in range(3):
                    y = y * (1.5 - 0.5 * d2c * y * y)
                dist = d2 * y
                sim = jnp.exp(dist)
                sig = 1.0 / (1.0 + jnp.exp(-sim))
                eb = gb + sgb
                out_v[pl.ds(eb, 16)] = ev_v[pl.ds(eb, 16)] * sig
                return c

            lax.fori_loop(0, _G // 16, subgroup, 0)

        issue(0, 0)
        issue(1, 1)

        def outer(tt, carry):
            g0 = tt * 3
            for k in range(3):
                issue(g0 + k + 2, (k + 2) % 3)
                compute(g0 + k, k)
            return carry

        lax.fori_loop(0, (_NG - 2) // 3, outer, 0)
        compute(_NG - 2, 0)
        compute(_NG - 1, 1)
        pltpu.sync_copy(out_v, out_hbm.at[pl.ds(base, _EPW)])

    return body(u_tab, v_tab, src, dst, ev)


def kernel(Eu, Ev, W1, b1, W2, b2, edge_index, edge_val):
    u = _transform(Eu, W1, b1)
    v = _transform(Ev, W2, b2)
    u_pk = lax.bitcast_convert_type(u.reshape(_N, _DW, 2), jnp.float32)
    v_pk = lax.bitcast_convert_type(v.reshape(_N, _DW, 2), jnp.float32)
    return _edge_values(u_pk, v_pk, edge_index[0], edge_index[1], edge_val)


# speedup vs baseline: 1.0914x; 1.0914x over previous
import functools

import jax
import jax.numpy as jnp
from jax import lax
from jax.experimental import pallas as pl
from jax.experimental.pallas import tpu as pltpu
from jax.experimental.pallas import tpu_sc as plsc

_N = 10000
_D = 128
_E = 320000
_NW = 32
_EPW = _E // _NW
_G = 96
_NGF = _EPW // _G
_TB = _NGF * _G


def _mlp_block_pos(x_ref, w_ref, b_ref, o_ref):
    y = jnp.dot(x_ref[...], w_ref[...], preferred_element_type=jnp.float32)
    o_ref[...] = jnp.maximum(y + b_ref[...], 0.0)


def _mlp_block_neg(x_ref, w_ref, b_ref, o_ref):
    y = jnp.dot(x_ref[...], w_ref[...], preferred_element_type=jnp.float32)
    o_ref[...] = jnp.minimum(-y - b_ref[...], 0.0)


def _transform(x, w, b, body, bl=2000):
    n, d = x.shape
    return pl.pallas_call(
        body,
        grid=(n // bl,),
        in_specs=[
            pl.BlockSpec((bl, d), lambda i: (i, 0)),
            pl.BlockSpec((d, d), lambda i: (0, 0)),
            pl.BlockSpec((1, d), lambda i: (0, 0)),
        ],
        out_specs=pl.BlockSpec((bl, d), lambda i: (i, 0)),
        out_shape=jax.ShapeDtypeStruct((n, d), jnp.float32),
    )(x, w, b.reshape(1, d))


def _edge_values(u_tab, vneg_tab, src, dst, ev):
    mesh = plsc.VectorSubcoreMesh(core_axis_name="c", subcore_axis_name="s")

    @functools.partial(
        pl.kernel,
        mesh=mesh,
        out_type=jax.ShapeDtypeStruct((_E,), jnp.float32),
        compiler_params=pltpu.CompilerParams(needs_layout_passes=False),
        scratch_types=[
            pltpu.VMEM((_EPW,), jnp.int32),
            pltpu.VMEM((_EPW,), jnp.int32),
            pltpu.VMEM((_EPW,), jnp.float32),
            pltpu.VMEM((_EPW,), jnp.float32),
            pltpu.VMEM((3, _G, _D), jnp.float32),
            pltpu.SemaphoreType.DMA,
            pltpu.SemaphoreType.DMA,
            pltpu.SemaphoreType.DMA,
            pltpu.SemaphoreType.DMA,
            pltpu.SemaphoreType.DMA,
            pltpu.SemaphoreType.DMA,
        ],
    )
    def body(u_hbm, v_hbm, src_hbm, dst_hbm, ev_hbm, out_hbm,
             src_v, dst_v, ev_v, out_v, du_rows,
             su0, su1, su2, sv0, sv1, sv2):
        wid = lax.axis_index("s") * 2 + lax.axis_index("c")
        base = wid * _EPW
        pltpu.sync_copy(src_hbm.at[pl.ds(base, _EPW)], src_v)
        pltpu.sync_copy(dst_hbm.at[pl.ds(base, _EPW)], dst_v)
        pltpu.sync_copy(ev_hbm.at[pl.ds(base, _EPW)], ev_v)

        lanes = lax.iota(jnp.int32, 16)
        sem_u = (su0, su1, su2)
        sem_v = (sv0, sv1, sv2)

        def issue_u(g, b, n=_G):
            pltpu.async_copy(u_hbm.at[src_v.at[pl.ds(g * _G, n)]],
                             du_rows.at[b].at[pl.ds(0, n)], sem_u[b])

        def wait_u(g, b, n=_G):
            pltpu.make_async_copy(u_hbm.at[src_v.at[pl.ds(g * _G, n)]],
                                  du_rows.at[b].at[pl.ds(0, n)],
                                  sem_u[b]).wait()

        def issue_vadd(g, b, n=_G):
            pltpu.async_copy(v_hbm.at[dst_v.at[pl.ds(g * _G, n)]],
                             du_rows.at[b].at[pl.ds(0, n)], sem_v[b],
                             add=True)

        def wait_vadd(g, b, n=_G):
            pltpu.make_async_copy(v_hbm.at[dst_v.at[pl.ds(g * _G, n)]],
                                  du_rows.at[b].at[pl.ds(0, n)],
                                  sem_v[b]).wait()

        def sub16(b, eb, sgb):
            d2 = jnp.zeros((16,), jnp.float32)
            for k in range(16):
                e = sgb + k
                acc = jnp.zeros((16,), jnp.float32)
                for j in range(_D // 16):
                    du = du_rows[b, e, pl.ds(j * 16, 16)]
                    acc = acc + du * du
                d2 = jnp.where(lanes == k, jnp.sum(acc), d2)
            d2c = jnp.maximum(d2, 1e-30)
            bi = lax.bitcast_convert_type(d2c, jnp.int32)
            bi = 0x5F3759DF - lax.shift_right_arithmetic(bi, 1)
            y = lax.bitcast_convert_type(bi, jnp.float32)
            for _ in range(3):
                y = y * (1.5 - 0.5 * d2c * y * y)
            dist = d2 * y
            sim = jnp.exp(dist)
            sig = 1.0 / (1.0 + jnp.exp(-sim))
            out_v[pl.ds(eb, 16)] = ev_v[pl.ds(eb, 16)] * sig

        def compute(g, b):
            gb = g * _G

            def subgroup(sg, c):
                sub16(b, gb + sg * 16, sg * 16)
                return c

            lax.fori_loop(0, _G // 16, subgroup, 0)

        issue_u(0, 0)
        wait_u(0, 0)
        issue_vadd(0, 0)
        issue_u(1, 1)

        def outer(tt, carry):
            g0 = tt * 3
            for k in range(3):
                g = g0 + k
                issue_u(g + 2, (k + 2) % 3)
                wait_u(g + 1, (k + 1) % 3)
                issue_vadd(g + 1, (k + 1) % 3)
                wait_vadd(g, k)
                compute(g, k)
            return carry

        lax.fori_loop(0, (_NGF - 2) // 3, outer, 0)
        wait_u(_NGF - 1, 1)
        issue_vadd(_NGF - 1, 1)
        issue_u(_NGF, 2, n=16)
        wait_vadd(_NGF - 2, 0)
        compute(_NGF - 2, 0)
        wait_u(_NGF, 2, n=16)
        issue_vadd(_NGF, 2, n=16)
        wait_vadd(_NGF - 1, 1)
        compute(_NGF - 1, 1)
        wait_vadd(_NGF, 2, n=16)
        sub16(2, _TB, 0)

        pltpu.sync_copy(out_v, out_hbm.at[pl.ds(base, _EPW)])

    return body(u_tab, vneg_tab, src, dst, ev)


def kernel(Eu, Ev, W1, b1, W2, b2, edge_index, edge_val):
    u = _transform(Eu, W1, b1, _mlp_block_pos)
    vneg = _transform(Ev, W2, b2, _mlp_block_neg)
    return _edge_values(u, vneg, edge_index[0], edge_index[1], edge_val)


# speedup vs baseline: 1.1200x; 1.0262x over previous
import functools

import jax
import jax.numpy as jnp
from jax import lax
from jax.experimental import pallas as pl
from jax.experimental.pallas import tpu as pltpu
from jax.experimental.pallas import tpu_sc as plsc

_N = 10000
_D = 128
_E = 320000
_NW = 32
_EPW = _E // _NW
_G = 96
_NGF = _EPW // _G
_TB = _NGF * _G


def _mlp_both_block(eu_ref, w1_ref, b1_ref, ev_ref, w2_ref, b2_ref,
                    u_ref, vn_ref):
    yu = jnp.dot(eu_ref[...], w1_ref[...], preferred_element_type=jnp.float32)
    u_ref[...] = jnp.maximum(yu + b1_ref[...], 0.0)
    yv = jnp.dot(ev_ref[...], w2_ref[...], preferred_element_type=jnp.float32)
    vn_ref[...] = jnp.minimum(-yv - b2_ref[...], 0.0)


def _transform_both(eu, w1, b1, ev, w2, b2, bl=2000):
    n, d = eu.shape
    row_spec = pl.BlockSpec((bl, d), lambda i: (i, 0))
    w_spec = pl.BlockSpec((d, d), lambda i: (0, 0))
    b_spec = pl.BlockSpec((1, d), lambda i: (0, 0))
    return pl.pallas_call(
        _mlp_both_block,
        grid=(n // bl,),
        in_specs=[row_spec, w_spec, b_spec, row_spec, w_spec, b_spec],
        out_specs=(row_spec, row_spec),
        out_shape=(jax.ShapeDtypeStruct((n, d), jnp.float32),
                   jax.ShapeDtypeStruct((n, d), jnp.float32)),
    )(eu, w1, b1.reshape(1, d), ev, w2, b2.reshape(1, d))


def _edge_values(u_tab, vneg_tab, src, dst, ev):
    mesh = plsc.VectorSubcoreMesh(core_axis_name="c", subcore_axis_name="s")

    @functools.partial(
        pl.kernel,
        mesh=mesh,
        out_type=jax.ShapeDtypeStruct((_E,), jnp.float32),
        compiler_params=pltpu.CompilerParams(needs_layout_passes=False),
        scratch_types=[
            pltpu.VMEM((_EPW,), jnp.int32),
            pltpu.VMEM((_EPW,), jnp.int32),
            pltpu.VMEM((_EPW,), jnp.float32),
            pltpu.VMEM((_EPW,), jnp.float32),
            pltpu.VMEM((3, _G, _D), jnp.float32),
            pltpu.SemaphoreType.DMA,
            pltpu.SemaphoreType.DMA,
            pltpu.SemaphoreType.DMA,
            pltpu.SemaphoreType.DMA,
            pltpu.SemaphoreType.DMA,
            pltpu.SemaphoreType.DMA,
        ],
    )
    def body(u_hbm, v_hbm, src_hbm, dst_hbm, ev_hbm, out_hbm,
             src_v, dst_v, ev_v, out_v, du_rows,
             su0, su1, su2, sv0, sv1, sv2):
        wid = lax.axis_index("s") * 2 + lax.axis_index("c")
        base = wid * _EPW
        pltpu.sync_copy(src_hbm.at[pl.ds(base, _EPW)], src_v)
        pltpu.sync_copy(dst_hbm.at[pl.ds(base, _EPW)], dst_v)
        pltpu.sync_copy(ev_hbm.at[pl.ds(base, _EPW)], ev_v)

        lanes = lax.iota(jnp.int32, 16)
        sem_u = (su0, su1, su2)
        sem_v = (sv0, sv1, sv2)

        def issue_u(g, b, n=_G):
            pltpu.async_copy(u_hbm.at[src_v.at[pl.ds(g * _G, n)]],
                             du_rows.at[b].at[pl.ds(0, n)], sem_u[b])

        def wait_u(g, b, n=_G):
            pltpu.make_async_copy(u_hbm.at[src_v.at[pl.ds(g * _G, n)]],
                                  du_rows.at[b].at[pl.ds(0, n)],
                                  sem_u[b]).wait()

        def issue_vadd(g, b, n=_G):
            pltpu.async_copy(v_hbm.at[dst_v.at[pl.ds(g * _G, n)]],
                             du_rows.at[b].at[pl.ds(0, n)], sem_v[b],
                             add=True)

        def wait_vadd(g, b, n=_G):
            pltpu.make_async_copy(v_hbm.at[dst_v.at[pl.ds(g * _G, n)]],
                                  du_rows.at[b].at[pl.ds(0, n)],
                                  sem_v[b]).wait()

        def sub16(b, eb, sgb):
            d2 = jnp.zeros((16,), jnp.float32)
            for k in range(16):
                e = sgb + k
                acc = jnp.zeros((16,), jnp.float32)
                for j in range(_D // 16):
                    du = du_rows[b, e, pl.ds(j * 16, 16)]
                    acc = acc + du * du
                d2 = jnp.where(lanes == k, jnp.sum(acc), d2)
            d2c = jnp.maximum(d2, 1e-30)
            bi = lax.bitcast_convert_type(d2c, jnp.int32)
            bi = 0x5F3759DF - lax.shift_right_arithmetic(bi, 1)
            y = lax.bitcast_convert_type(bi, jnp.float32)
            for _ in range(3):
                y = y * (1.5 - 0.5 * d2c * y * y)
            dist = d2 * y
            sim = jnp.exp(dist)
            sig = 1.0 / (1.0 + jnp.exp(-sim))
            out_v[pl.ds(eb, 16)] = ev_v[pl.ds(eb, 16)] * sig

        def compute(g, b):
            gb = g * _G

            def subgroup(sg, c):
                sub16(b, gb + sg * 16, sg * 16)
                return c

            lax.fori_loop(0, _G // 16, subgroup, 0)

        issue_u(0, 0)
        wait_u(0, 0)
        issue_vadd(0, 0)
        issue_u(1, 1)

        def outer(tt, carry):
            g0 = tt * 3
            for k in range(3):
                g = g0 + k
                issue_u(g + 2, (k + 2) % 3)
                wait_u(g + 1, (k + 1) % 3)
                issue_vadd(g + 1, (k + 1) % 3)
                wait_vadd(g, k)
                compute(g, k)
            return carry

        lax.fori_loop(0, (_NGF - 2) // 3, outer, 0)
        wait_u(_NGF - 1, 1)
        issue_vadd(_NGF - 1, 1)
        issue_u(_NGF, 2, n=16)
        wait_vadd(_NGF - 2, 0)
        compute(_NGF - 2, 0)
        wait_u(_NGF, 2, n=16)
        issue_vadd(_NGF, 2, n=16)
        wait_vadd(_NGF - 1, 1)
        compute(_NGF - 1, 1)
        wait_vadd(_NGF, 2, n=16)
        sub16(2, _TB, 0)

        pltpu.sync_copy(out_v, out_hbm.at[pl.ds(base, _EPW)])

    return body(u_tab, vneg_tab, src, dst, ev)


def kernel(Eu, Ev, W1, b1, W2, b2, edge_index, edge_val):
    u, vneg = _transform_both(Eu, W1, b1, Ev, W2, b2)
    return _edge_values(u, vneg, edge_index[0], edge_index[1], edge_val)


# speedup vs baseline: 1.1822x; 1.0555x over previous
import functools

import jax
import jax.numpy as jnp
from jax import lax
from jax.experimental import pallas as pl
from jax.experimental.pallas import tpu as pltpu
from jax.experimental.pallas import tpu_sc as plsc

_N = 10000
_D = 128
_E = 320000
_NW = 32
_EPW = _E // _NW
_G = 96
_NGF = _EPW // _G
_TB = _NGF * _G


def _mlp_both_block(eu_ref, w1_ref, b1_ref, ev_ref, w2_ref, b2_ref,
                    u_ref, vn_ref):
    yu = jnp.dot(eu_ref[...], w1_ref[...], preferred_element_type=jnp.float32)
    u_ref[...] = jnp.maximum(yu + b1_ref[...], 0.0)
    yv = jnp.dot(ev_ref[...], w2_ref[...], preferred_element_type=jnp.float32)
    vn_ref[...] = jnp.minimum(-yv - b2_ref[...], 0.0)


def _transform_both(eu, w1, b1, ev, w2, b2, bl=2000):
    n, d = eu.shape
    row_spec = pl.BlockSpec((bl, d), lambda i: (i, 0))
    w_spec = pl.BlockSpec((d, d), lambda i: (0, 0))
    b_spec = pl.BlockSpec((1, d), lambda i: (0, 0))
    return pl.pallas_call(
        _mlp_both_block,
        grid=(n // bl,),
        in_specs=[row_spec, w_spec, b_spec, row_spec, w_spec, b_spec],
        out_specs=(row_spec, row_spec),
        out_shape=(jax.ShapeDtypeStruct((n, d), jnp.float32),
                   jax.ShapeDtypeStruct((n, d), jnp.float32)),
    )(eu, w1, b1.reshape(1, d), ev, w2, b2.reshape(1, d))


def _edge_values(u_tab, vneg_tab, edge_flat, ev):
    mesh = plsc.VectorSubcoreMesh(core_axis_name="c", subcore_axis_name="s")

    @functools.partial(
        pl.kernel,
        mesh=mesh,
        out_type=jax.ShapeDtypeStruct((_E,), jnp.float32),
        compiler_params=pltpu.CompilerParams(needs_layout_passes=False),
        scratch_types=[
            pltpu.VMEM((_EPW,), jnp.int32),
            pltpu.VMEM((_EPW,), jnp.int32),
            pltpu.VMEM((_EPW,), jnp.float32),
            pltpu.VMEM((_EPW,), jnp.float32),
            pltpu.VMEM((3, _G, _D), jnp.float32),
            pltpu.SemaphoreType.DMA,
            pltpu.SemaphoreType.DMA,
            pltpu.SemaphoreType.DMA,
            pltpu.SemaphoreType.DMA,
            pltpu.SemaphoreType.DMA,
            pltpu.SemaphoreType.DMA,
        ],
    )
    def body(u_hbm, v_hbm, edge_hbm, ev_hbm, out_hbm,
             src_v, dst_v, ev_v, out_v, du_rows,
             su0, su1, su2, sv0, sv1, sv2):
        wid = lax.axis_index("s") * 2 + lax.axis_index("c")
        base = wid * _EPW
        pltpu.sync_copy(edge_hbm.at[pl.ds(base, _EPW)], src_v)
        pltpu.sync_copy(edge_hbm.at[pl.ds(_E + base, _EPW)], dst_v)
        pltpu.sync_copy(ev_hbm.at[pl.ds(base, _EPW)], ev_v)

        lanes = lax.iota(jnp.int32, 16)
        sem_u = (su0, su1, su2)
        sem_v = (sv0, sv1, sv2)

        def issue_u(g, b, n=_G):
            pltpu.async_copy(u_hbm.at[src_v.at[pl.ds(g * _G, n)]],
                             du_rows.at[b].at[pl.ds(0, n)], sem_u[b])

        def wait_u(g, b, n=_G):
            pltpu.make_async_copy(u_hbm.at[src_v.at[pl.ds(g * _G, n)]],
                                  du_rows.at[b].at[pl.ds(0, n)],
                                  sem_u[b]).wait()

        def issue_vadd(g, b, n=_G):
            pltpu.async_copy(v_hbm.at[dst_v.at[pl.ds(g * _G, n)]],
                             du_rows.at[b].at[pl.ds(0, n)], sem_v[b],
                             add=True)

        def wait_vadd(g, b, n=_G):
            pltpu.make_async_copy(v_hbm.at[dst_v.at[pl.ds(g * _G, n)]],
                                  du_rows.at[b].at[pl.ds(0, n)],
                                  sem_v[b]).wait()

        def sub16(b, eb, sgb):
            d2 = jnp.zeros((16,), jnp.float32)
            for k in range(16):
                e = sgb + k
                acc = jnp.zeros((16,), jnp.float32)
                for j in range(_D // 16):
                    du = du_rows[b, e, pl.ds(j * 16, 16)]
                    acc = acc + du * du
                d2 = jnp.where(lanes == k, jnp.sum(acc), d2)
            d2c = jnp.maximum(d2, 1e-30)
            bi = lax.bitcast_convert_type(d2c, jnp.int32)
            bi = 0x5F3759DF - lax.shift_right_arithmetic(bi, 1)
            y = lax.bitcast_convert_type(bi, jnp.float32)
            for _ in range(3):
                y = y * (1.5 - 0.5 * d2c * y * y)
            dist = d2 * y
            sim = jnp.exp(dist)
            sig = 1.0 / (1.0 + jnp.exp(-sim))
            out_v[pl.ds(eb, 16)] = ev_v[pl.ds(eb, 16)] * sig

        def compute(g, b):
            gb = g * _G

            def subgroup(sg, c):
                sub16(b, gb + sg * 16, sg * 16)
                return c

            lax.fori_loop(0, _G // 16, subgroup, 0)

        issue_u(0, 0)
        wait_u(0, 0)
        issue_vadd(0, 0)
        issue_u(1, 1)

        def outer(tt, carry):
            g0 = tt * 3
            for k in range(3):
                g = g0 + k
                issue_u(g + 2, (k + 2) % 3)
                wait_u(g + 1, (k + 1) % 3)
                issue_vadd(g + 1, (k + 1) % 3)
                wait_vadd(g, k)
                compute(g, k)
            return carry

        lax.fori_loop(0, (_NGF - 2) // 3, outer, 0)
        wait_u(_NGF - 1, 1)
        issue_vadd(_NGF - 1, 1)
        issue_u(_NGF, 2, n=16)
        wait_vadd(_NGF - 2, 0)
        compute(_NGF - 2, 0)
        wait_u(_NGF, 2, n=16)
        issue_vadd(_NGF, 2, n=16)
        wait_vadd(_NGF - 1, 1)
        compute(_NGF - 1, 1)
        wait_vadd(_NGF, 2, n=16)
        sub16(2, _TB, 0)

        pltpu.sync_copy(out_v, out_hbm.at[pl.ds(base, _EPW)])

    return body(u_tab, vneg_tab, edge_flat, ev)


def kernel(Eu, Ev, W1, b1, W2, b2, edge_index, edge_val):
    u, vneg = _transform_both(Eu, W1, b1, Ev, W2, b2)
    return _edge_values(u, vneg, edge_index.reshape(2 * _E), edge_val)


# speedup vs baseline: 1.1858x; 1.0031x over previous
import functools

import jax
import jax.numpy as jnp
from jax import lax
from jax.experimental import pallas as pl
from jax.experimental.pallas import tpu as pltpu
from jax.experimental.pallas import tpu_sc as plsc

_N = 10000
_D = 128
_E = 320000
_NW = 32
_EPW = _E // _NW
_G = 96
_NGF = _EPW // _G
_TB = _NGF * _G


def _mlp_both_block(eu_ref, w1_ref, b1_ref, ev_ref, w2_ref, b2_ref,
                    u_ref, vn_ref):
    yu = jnp.dot(eu_ref[...], w1_ref[...], preferred_element_type=jnp.float32)
    u_ref[...] = jnp.maximum(yu + b1_ref[...], 0.0)
    yv = jnp.dot(ev_ref[...], w2_ref[...], preferred_element_type=jnp.float32)
    vn_ref[...] = jnp.minimum(-yv - b2_ref[...], 0.0)


def _transform_both(eu, w1, b1, ev, w2, b2, bl=2000):
    n, d = eu.shape
    row_spec = pl.BlockSpec((bl, d), lambda i: (i, 0))
    w_spec = pl.BlockSpec((d, d), lambda i: (0, 0))
    b_spec = pl.BlockSpec((1, d), lambda i: (0, 0))
    return pl.pallas_call(
        _mlp_both_block,
        grid=(n // bl,),
        in_specs=[row_spec, w_spec, b_spec, row_spec, w_spec, b_spec],
        out_specs=(row_spec, row_spec),
        out_shape=(jax.ShapeDtypeStruct((n, d), jnp.float32),
                   jax.ShapeDtypeStruct((n, d), jnp.float32)),
    )(eu, w1, b1.reshape(1, d), ev, w2, b2.reshape(1, d))


def _edge_values(u_tab, vneg_tab, edge_flat, ev):
    mesh = plsc.VectorSubcoreMesh(core_axis_name="c", subcore_axis_name="s")

    @functools.partial(
        pl.kernel,
        mesh=mesh,
        out_type=jax.ShapeDtypeStruct((_E,), jnp.float32),
        compiler_params=pltpu.CompilerParams(needs_layout_passes=False),
        scratch_types=[
            pltpu.VMEM((_EPW,), jnp.int32),
            pltpu.VMEM((_EPW,), jnp.int32),
            pltpu.VMEM((_EPW,), jnp.float32),
            pltpu.VMEM((_EPW,), jnp.float32),
            pltpu.VMEM((3, _G, _D), jnp.float32),
            pltpu.SemaphoreType.DMA,
            pltpu.SemaphoreType.DMA,
            pltpu.SemaphoreType.DMA,
            pltpu.SemaphoreType.DMA,
            pltpu.SemaphoreType.DMA,
            pltpu.SemaphoreType.DMA,
        ],
    )
    def body(u_hbm, v_hbm, edge_hbm, ev_hbm, out_hbm,
             src_v, dst_v, ev_v, out_v, du_rows,
             su0, su1, su2, sv0, sv1, sv2):
        wid = lax.axis_index("s") * 2 + lax.axis_index("c")
        base = wid * _EPW
        pltpu.sync_copy(edge_hbm.at[pl.ds(base, _EPW)], src_v)

        lanes = lax.iota(jnp.int32, 16)
        sem_u = (su0, su1, su2)
        sem_v = (sv0, sv1, sv2)

        def issue_u(g, b, n=_G):
            pltpu.async_copy(u_hbm.at[src_v.at[pl.ds(g * _G, n)]],
                             du_rows.at[b].at[pl.ds(0, n)], sem_u[b])

        def wait_u(g, b, n=_G):
            pltpu.make_async_copy(u_hbm.at[src_v.at[pl.ds(g * _G, n)]],
                                  du_rows.at[b].at[pl.ds(0, n)],
                                  sem_u[b]).wait()

        def issue_vadd(g, b, n=_G):
            pltpu.async_copy(v_hbm.at[dst_v.at[pl.ds(g * _G, n)]],
                             du_rows.at[b].at[pl.ds(0, n)], sem_v[b],
                             add=True)

        def wait_vadd(g, b, n=_G):
            pltpu.make_async_copy(v_hbm.at[dst_v.at[pl.ds(g * _G, n)]],
                                  du_rows.at[b].at[pl.ds(0, n)],
                                  sem_v[b]).wait()

        def sub16(b, eb, sgb):
            d2 = jnp.zeros((16,), jnp.float32)
            for k in range(16):
                e = sgb + k
                acc = jnp.zeros((16,), jnp.float32)
                for j in range(_D // 16):
                    du = du_rows[b, e, pl.ds(j * 16, 16)]
                    acc = acc + du * du
                d2 = jnp.where(lanes == k, jnp.sum(acc), d2)
            d2c = jnp.maximum(d2, 1e-30)
            bi = lax.bitcast_convert_type(d2c, jnp.int32)
            bi = 0x5F3759DF - lax.shift_right_arithmetic(bi, 1)
            y = lax.bitcast_convert_type(bi, jnp.float32)
            for _ in range(3):
                y = y * (1.5 - 0.5 * d2c * y * y)
            dist = d2 * y
            sim = jnp.exp(dist)
            sig = 1.0 / (1.0 + jnp.exp(-sim))
            out_v[pl.ds(eb, 16)] = ev_v[pl.ds(eb, 16)] * sig

        def compute(g, b):
            gb = g * _G

            def subgroup(sg, c):
                sub16(b, gb + sg * 16, sg * 16)
                return c

            lax.fori_loop(0, _G // 16, subgroup, 0)

        issue_u(0, 0)
        pltpu.sync_copy(edge_hbm.at[pl.ds(_E + base, _EPW)], dst_v)
        pltpu.sync_copy(ev_hbm.at[pl.ds(base, _EPW)], ev_v)
        wait_u(0, 0)
        issue_vadd(0, 0)
        issue_u(1, 1)

        def outer(tt, carry):
            g0 = tt * 3
            for k in range(3):
                g = g0 + k
                issue_u(g + 2, (k + 2) % 3)
                wait_u(g + 1, (k + 1) % 3)
                issue_vadd(g + 1, (k + 1) % 3)
                wait_vadd(g, k)
                compute(g, k)
            return carry

        lax.fori_loop(0, (_NGF - 2) // 3, outer, 0)
        wait_u(_NGF - 1, 1)
        issue_vadd(_NGF - 1, 1)
        issue_u(_NGF, 2, n=16)
        wait_vadd(_NGF - 2, 0)
        compute(_NGF - 2, 0)
        wait_u(_NGF, 2, n=16)
        issue_vadd(_NGF, 2, n=16)
        wait_vadd(_NGF - 1, 1)
        compute(_NGF - 1, 1)
        wait_vadd(_NGF, 2, n=16)
        sub16(2, _TB, 0)

        pltpu.sync_copy(out_v, out_hbm.at[pl.ds(base, _EPW)])

    return body(u_tab, vneg_tab, edge_flat, ev)


def kernel(Eu, Ev, W1, b1, W2, b2, edge_index, edge_val):
    u, vneg = _transform_both(Eu, W1, b1, Ev, W2, b2)
    return _edge_values(u, vneg, edge_index.reshape(2 * _E), edge_val)


# speedup vs baseline: 1.5139x; 1.2767x over previous
import functools

import jax
import jax.numpy as jnp
from jax import lax
from jax.experimental import pallas as pl
from jax.experimental.pallas import tpu as pltpu
from jax.experimental.pallas import tpu_sc as plsc

_N = 10000
_D = 128
_E = 320000
_NW = 32
_EPW = _E // _NW
_G = 96
_NGF = _EPW // _G
_TB = _NGF * _G


def _pack_bf16_halves(y):
    yb = lax.bitcast_convert_type(y, jnp.int32)
    r = yb + 0x7FFF + (lax.shift_right_logical(yb, 16) & 1)
    lo = lax.shift_right_logical(r[:, : _D // 2], 16)
    hi = r[:, _D // 2:] & jnp.int32(-65536)
    return lo | hi


def _mlp_both_block(eu_ref, w1_ref, b1_ref, ev_ref, w2_ref, b2_ref,
                    u_ref, v_ref):
    yu = jnp.dot(eu_ref[...], w1_ref[...], preferred_element_type=jnp.float32)
    u_ref[...] = _pack_bf16_halves(jnp.maximum(yu + b1_ref[...], 0.0))
    yv = jnp.dot(ev_ref[...], w2_ref[...], preferred_element_type=jnp.float32)
    v_ref[...] = _pack_bf16_halves(jnp.maximum(yv + b2_ref[...], 0.0))


def _transform_both(eu, w1, b1, ev, w2, b2, bl=2000):
    n, d = eu.shape
    row_spec = pl.BlockSpec((bl, d), lambda i: (i, 0))
    w_spec = pl.BlockSpec((d, d), lambda i: (0, 0))
    b_spec = pl.BlockSpec((1, d), lambda i: (0, 0))
    pk_spec = pl.BlockSpec((bl, d // 2), lambda i: (i, 0))
    return pl.pallas_call(
        _mlp_both_block,
        grid=(n // bl,),
        in_specs=[row_spec, w_spec, b_spec, row_spec, w_spec, b_spec],
        out_specs=(pk_spec, pk_spec),
        out_shape=(jax.ShapeDtypeStruct((n, d // 2), jnp.int32),
                   jax.ShapeDtypeStruct((n, d // 2), jnp.int32)),
    )(eu, w1, b1.reshape(1, d), ev, w2, b2.reshape(1, d))


def _edge_values(u_tab, vneg_tab, edge_flat, ev):
    mesh = plsc.VectorSubcoreMesh(core_axis_name="c", subcore_axis_name="s")

    @functools.partial(
        pl.kernel,
        mesh=mesh,
        out_type=jax.ShapeDtypeStruct((_E,), jnp.float32),
        compiler_params=pltpu.CompilerParams(
            needs_layout_passes=False, use_tc_tiling_on_sc=False),
        scratch_types=[
            pltpu.VMEM((_EPW,), jnp.int32),
            pltpu.VMEM((_EPW,), jnp.int32),
            pltpu.VMEM((_EPW,), jnp.float32),
            pltpu.VMEM((_EPW,), jnp.float32),
            pltpu.VMEM((3, _G, _D // 2), jnp.int32),
            pltpu.VMEM((3, _G, _D // 2), jnp.int32),
            pltpu.SemaphoreType.DMA,
            pltpu.SemaphoreType.DMA,
            pltpu.SemaphoreType.DMA,
            pltpu.SemaphoreType.DMA,
            pltpu.SemaphoreType.DMA,
            pltpu.SemaphoreType.DMA,
        ],
    )
    def body(u_hbm, v_hbm, edge_hbm, ev_hbm, out_hbm,
             src_v, dst_v, ev_v, out_v, u_rows, v_rows,
             su0, su1, su2, sv0, sv1, sv2):
        wid = lax.axis_index("s") * 2 + lax.axis_index("c")
        base = wid * _EPW
        pltpu.sync_copy(edge_hbm.at[pl.ds(base, _EPW)], src_v)

        lanes = lax.iota(jnp.int32, 16)
        sem_u = (su0, su1, su2)
        sem_v = (sv0, sv1, sv2)

        def issue(g, b, n=_G):
            pltpu.async_copy(u_hbm.at[src_v.at[pl.ds(g * _G, n)]],
                             u_rows.at[b].at[pl.ds(0, n)], sem_u[b])
            pltpu.async_copy(v_hbm.at[dst_v.at[pl.ds(g * _G, n)]],
                             v_rows.at[b].at[pl.ds(0, n)], sem_v[b])

        def wait(g, b, n=_G):
            pltpu.make_async_copy(u_hbm.at[src_v.at[pl.ds(g * _G, n)]],
                                  u_rows.at[b].at[pl.ds(0, n)],
                                  sem_u[b]).wait()
            pltpu.make_async_copy(v_hbm.at[dst_v.at[pl.ds(g * _G, n)]],
                                  v_rows.at[b].at[pl.ds(0, n)],
                                  sem_v[b]).wait()

        def sub16(b, eb, sgb):
            d2 = jnp.zeros((16,), jnp.float32)
            for k in range(16):
                e = sgb + k
                acc = jnp.zeros((16,), jnp.float32)
                for j in range(_D // 32):
                    uw = u_rows[b, e, pl.ds(j * 16, 16)]
                    vw = v_rows[b, e, pl.ds(j * 16, 16)]
                    du = (plsc.bitcast(uw, jnp.bfloat16)
                          - plsc.bitcast(vw, jnp.bfloat16))
                    lo, hi = plsc.unpack(
                        du, format=plsc.PackFormat.INTERLEAVED)
                    acc = acc + lo * lo + hi * hi
                d2 = jnp.where(lanes == k, jnp.sum(acc), d2)
            d2c = jnp.maximum(d2, 1e-30)
            bi = lax.bitcast_convert_type(d2c, jnp.int32)
            bi = 0x5F3759DF - lax.shift_right_arithmetic(bi, 1)
            y = lax.bitcast_convert_type(bi, jnp.float32)
            for _ in range(3):
                y = y * (1.5 - 0.5 * d2c * y * y)
            dist = d2 * y
            sim = jnp.exp(dist)
            sig = 1.0 / (1.0 + jnp.exp(-sim))
            out_v[pl.ds(eb, 16)] = ev_v[pl.ds(eb, 16)] * sig

        def compute(g, b):
            gb = g * _G

            def subgroup(sg, c):
                sub16(b, gb + sg * 16, sg * 16)
                return c

            lax.fori_loop(0, _G // 16, subgroup, 0)

        pltpu.sync_copy(edge_hbm.at[pl.ds(_E + base, _EPW)], dst_v)
        issue(0, 0)
        pltpu.sync_copy(ev_hbm.at[pl.ds(base, _EPW)], ev_v)
        issue(1, 1)

        def outer(tt, carry):
            g0 = tt * 3
            for k in range(3):
                g = g0 + k
                issue(g + 2, (k + 2) % 3)
                wait(g, k)
                compute(g, k)
            return carry

        lax.fori_loop(0, (_NGF - 2) // 3, outer, 0)
        issue(_NGF, 2, n=16)
        wait(_NGF - 2, 0)
        compute(_NGF - 2, 0)
        wait(_NGF - 1, 1)
        compute(_NGF - 1, 1)
        wait(_NGF, 2, n=16)
        sub16(2, _TB, 0)

        pltpu.sync_copy(out_v, out_hbm.at[pl.ds(base, _EPW)])

    return body(u_tab, vneg_tab, edge_flat, ev)


def kernel(Eu, Ev, W1, b1, W2, b2, edge_index, edge_val):
    u, v = _transform_both(Eu, W1, b1, Ev, W2, b2)
    return _edge_values(u, v, edge_index.reshape(2 * _E), edge_val)
